# baseline (speedup 1.0000x reference)
import functools

import jax
import jax.numpy as jnp
from jax import lax
from jax.experimental import pallas as pl
from jax.experimental.pallas import tpu as pltpu
from jax.experimental.pallas import tpu_sc as plsc

N = 10000
E = 160000
ND = 256
ED = 16
H = 256

NC = 2
NS = 16
NW = NC * NS

C = 128
NCHUNK = E // C
HALF = N // NC
ZC = 40
DUM = 512


def _mm_kernel(a_ref, w_ref, o_ref):
    o_ref[...] = jnp.dot(a_ref[...], w_ref[...], preferred_element_type=jnp.float32)


def _mm_relu_kernel(a_ref, w_ref, o_ref):
    acc = jnp.dot(a_ref[...], w_ref[...], preferred_element_type=jnp.float32)
    o_ref[...] = jnp.maximum(acc, 0.0)


def _tc_matmul(a, w, relu=False, blk=2000, m=None):
    k = a.shape[1]
    m = a.shape[0] if m is None else m
    h = w.shape[1]
    return pl.pallas_call(
        _mm_relu_kernel if relu else _mm_kernel,
        grid=(m // blk,),
        in_specs=[pl.BlockSpec((blk, k), lambda i: (i, 0)),
                  pl.BlockSpec((k, h), lambda i: (0, 0))],
        out_specs=pl.BlockSpec((blk, h), lambda i: (i, 0)),
        out_shape=jax.ShapeDtypeStruct((m, h), jnp.float32),
    )(a, w)


def _msgs_kernel(g_ref, ea_ref, w_ref, o_ref):
    acc = jnp.dot(ea_ref[...], w_ref[...], preferred_element_type=jnp.float32)
    o_ref[...] = jnp.maximum(g_ref[...] + acc, 0.0)


def _tc_msgs(g, ea, w, blk=2000):
    return pl.pallas_call(
        _msgs_kernel,
        grid=(E // blk,),
        in_specs=[pl.BlockSpec((blk, H), lambda i: (i, 0)),
                  pl.BlockSpec((blk, ED), lambda i: (i, 0)),
                  pl.BlockSpec((ED, H), lambda i: (0, 0))],
        out_specs=pl.BlockSpec((blk, H), lambda i: (i, 0)),
        out_shape=jax.ShapeDtypeStruct((E, H), jnp.float32),
    )(g, ea, w)


def _ln_kernel(a_ref, b_ref, g_ref, bb_ref, o_ref):
    v = a_ref[...] + b_ref[...]
    mean = jnp.mean(v, axis=-1, keepdims=True)
    var = jnp.mean((v - mean) ** 2, axis=-1, keepdims=True)
    o_ref[...] = (v - mean) / jnp.sqrt(var + 1e-5) * g_ref[...] + bb_ref[...]


def _tc_ln(a, b, g, bb, blk=2000):
    return pl.pallas_call(
        _ln_kernel,
        grid=(E // blk,),
        in_specs=[pl.BlockSpec((blk, H), lambda i: (i, 0)),
                  pl.BlockSpec((blk, H), lambda i: (i, 0)),
                  pl.BlockSpec((1, H), lambda i: (0, 0)),
                  pl.BlockSpec((1, H), lambda i: (0, 0))],
        out_specs=pl.BlockSpec((blk, H), lambda i: (i, 0)),
        out_shape=jax.ShapeDtypeStruct((E, H), jnp.float32),
    )(a, b, g, bb)


def _out_kernel(x_ref, m_ref, wx_ref, wm_ref, b_ref, o_ref):
    acc = jnp.dot(x_ref[...], wx_ref[...], preferred_element_type=jnp.float32)
    acc = acc + jnp.dot(m_ref[...], wm_ref[...], preferred_element_type=jnp.float32)
    o_ref[...] = jnp.maximum(acc + b_ref[...], 0.0)


def _tc_out(x, m, wx, wm, b, blk=2000):
    return pl.pallas_call(
        _out_kernel,
        grid=(N // blk,),
        in_specs=[pl.BlockSpec((blk, ND), lambda i: (i, 0)),
                  pl.BlockSpec((blk, H), lambda i: (i, 0)),
                  pl.BlockSpec((ND, H), lambda i: (0, 0)),
                  pl.BlockSpec((H, H), lambda i: (0, 0)),
                  pl.BlockSpec((1, H), lambda i: (0, 0))],
        out_specs=pl.BlockSpec((blk, H), lambda i: (i, 0)),
        out_shape=jax.ShapeDtypeStruct((N, H), jnp.float32),
    )(x, m, wx, wm, b)


def _sc_gather(table, idx):
    mesh = plsc.VectorSubcoreMesh(core_axis_name="c", subcore_axis_name="s")
    base_cnt = NCHUNK // NW
    extra = NCHUNK - base_cnt * NW

    @functools.partial(
        pl.kernel,
        out_type=jax.ShapeDtypeStruct((E, H), jnp.float32),
        mesh=mesh,
        scratch_types=[
            pltpu.VMEM((C,), jnp.int32),
            pltpu.VMEM((C,), jnp.int32),
            pltpu.VMEM((C, H), jnp.float32),
            pltpu.VMEM((C, H), jnp.float32),
            pltpu.SemaphoreType.DMA,
            pltpu.SemaphoreType.DMA,
            pltpu.SemaphoreType.DMA,
            pltpu.SemaphoreType.DMA,
            pltpu.SemaphoreType.DMA,
            pltpu.SemaphoreType.DMA,
        ],
    )
    def k(table_hbm, idx_hbm, out_hbm, i0, i1, r0, r1,
          si0, si1, sg0, sg1, ss0, ss1):
        wid = lax.axis_index("s") * NC + lax.axis_index("c")
        cnt = base_cnt + (wid < extra).astype(jnp.int32)

        def cbase(j):
            return (wid + j * NW) * C

        pltpu.async_copy(idx_hbm.at[pl.ds(cbase(0), C)], i0, si0)

        def run_iter(j, ib, rb, sib, sgb, ssb, inxt, sinxt):
            pltpu.make_async_copy(idx_hbm.at[pl.ds(cbase(j), C)], ib, sib).wait()

            @pl.when(j >= 2)
            def _():
                pltpu.make_async_copy(
                    rb, out_hbm.at[pl.ds(cbase(j - 2), C)], ssb).wait()

            pltpu.async_copy(table_hbm.at[ib], rb, sgb)

            @pl.when(j + 1 < cnt)
            def _():
                pltpu.async_copy(
                    idx_hbm.at[pl.ds(cbase(j + 1), C)], inxt, sinxt)

            pltpu.make_async_copy(table_hbm.at[ib], rb, sgb).wait()
            pltpu.async_copy(rb, out_hbm.at[pl.ds(cbase(j), C)], ssb)

        def body(j, carry):
            lax.cond(
                j % 2 == 0,
                lambda jj: run_iter(jj, i0, r0, si0, sg0, ss0, i1, si1),
                lambda jj: run_iter(jj, i1, r1, si1, sg1, ss1, i0, si0),
                j,
            )
            return carry

        lax.fori_loop(0, cnt, body, 0)
        pltpu.make_async_copy(r0, out_hbm.at[pl.ds(0, C)], ss0).wait()
        pltpu.make_async_copy(r1, out_hbm.at[pl.ds(0, C)], ss1).wait()

    return k(table, idx)


OWN = 312
ACC = 336
PAD_ROW = ACC - 1
BIGC = 1600
NBIG = E // BIGC
CF = 64
PEND = 128


def _sc_scatter_add(vals, idx, zrows):
    mesh = plsc.VectorSubcoreMesh(core_axis_name="c", subcore_axis_name="s")

    @functools.partial(
        pl.kernel,
        out_type=(jax.ShapeDtypeStruct((N, H), jnp.float32),
                  jax.ShapeDtypeStruct((NW, E + CF), jnp.int32),
                  jax.ShapeDtypeStruct((NW, E + CF), jnp.int32),
                  jax.ShapeDtypeStruct((NW, 16), jnp.int32)),
        mesh=mesh,
        compiler_params=pltpu.CompilerParams(needs_layout_passes=False),
        scratch_types=[
            pltpu.VMEM((BIGC,), jnp.int32),
            pltpu.VMEM((BIGC,), jnp.int32),
            pltpu.VMEM((PEND,), jnp.int32),
            pltpu.VMEM((PEND,), jnp.int32),
            pltpu.VMEM((CF,), jnp.int32),
            pltpu.VMEM((CF,), jnp.int32),
            pltpu.VMEM((CF,), jnp.int32),
            pltpu.VMEM((CF,), jnp.int32),
            pltpu.VMEM((CF, H), jnp.float32),
            pltpu.VMEM((CF, H), jnp.float32),
            pltpu.VMEM((ACC, H), jnp.float32),
            pltpu.VMEM((16,), jnp.int32),
            pltpu.SemaphoreType.DMA,
            pltpu.SemaphoreType.DMA,
            pltpu.SemaphoreType.DMA,
            pltpu.SemaphoreType.DMA,
            pltpu.SemaphoreType.DMA,
            pltpu.SemaphoreType.DMA,
        ],
    )
    def k(vals_hbm, idx_hbm, z_hbm, out_hbm, le_hbm, la_hbm, cnt_hbm,
          sv0, sv1, ebuf, abuf, fid0, fid1, fadj0, fadj1, rows0, rows1,
          acc_v, cnt_v, sc0, sc1, sg0, sg1, sl0, sl1):
        c = lax.axis_index("c")
        s = lax.axis_index("s")
        wid = s * NC + c
        lane = lax.iota(jnp.int32, 16)
        base_row = wid * OWN
        ub = OWN + jnp.where(wid == NW - 1, ACC - OWN - 8, 0)

        for kz in range(8):
            pltpu.sync_copy(z_hbm, acc_v.at[pl.ds(kz * ZC, ZC)])
        pltpu.sync_copy(z_hbm.at[pl.ds(0, 16)], acc_v.at[pl.ds(320, 16)])

        def accumulate(fadjb, rowsb):
            def grp(gg, carry):
                va = fadjb[pl.ds(gg * 16, 16)]
                for i in range(16):
                    r = gg * 16 + i
                    ld = va[i]
                    for j in range(H // 16):
                        plsc.addupdate(acc_v.at[ld, pl.ds(j * 16, 16)],
                                       rowsb[r, pl.ds(j * 16, 16)])
                return carry

            lax.fori_loop(0, CF // 16, grp, 0)

        def fire_buf(o, fc, fidb, fadjb, rowsb, sgb, slb,
                     fido, fadjo, rowso, sgo):
            @pl.when(fc >= 2)
            def _():
                pltpu.make_async_copy(
                    fidb, le_hbm.at[wid, pl.ds(0, CF)], slb).wait()
                pltpu.make_async_copy(
                    fadjb, la_hbm.at[wid, pl.ds(0, CF)], slb).wait()

            for g in range(CF // 16):
                fidb[pl.ds(g * 16, 16)] = ebuf[pl.ds(g * 16, 16)]
                fadjb[pl.ds(g * 16, 16)] = abuf[pl.ds(g * 16, 16)]
            for g in range(CF // 16):
                ebuf[pl.ds(g * 16, 16)] = ebuf[pl.ds(CF + g * 16, 16)]
                abuf[pl.ds(g * 16, 16)] = abuf[pl.ds(CF + g * 16, 16)]

            pltpu.async_copy(fidb, le_hbm.at[wid, pl.ds(fc * CF, CF)], slb)
            pltpu.async_copy(fadjb, la_hbm.at[wid, pl.ds(fc * CF, CF)], slb)

            @pl.when(fc > 0)
            def _():
                pltpu.make_async_copy(vals_hbm.at[fido], rowso, sgo).wait()
                accumulate(fadjo, rowso)

            pltpu.async_copy(vals_hbm.at[fidb], rowsb, sgb)
            return (o - CF, fc + 1)

        def fire(state):
            return lax.cond(
                state[1] % 2 == 0,
                lambda st: fire_buf(st[0], st[1], fid0, fadj0, rows0, sg0, sl0,
                                    fid1, fadj1, rows1, sg1),
                lambda st: fire_buf(st[0], st[1], fid1, fadj1, rows1, sg1, sl1,
                                    fid0, fadj0, rows0, sg0),
                state)

        def maybe_fire(state):
            return lax.cond(state[0] >= CF, fire, lambda st: st, state)

        def inner(svb, ibase, state):
            def sub(gi, st):
                o, fc = st
                for g in range(4):
                    go = gi * 64 + g * 16
                    v = svb[pl.ds(go, 16)]
                    local = v - base_row
                    ok = (local >= 0) & (local < ub)

                    def compact(oo):
                        cs = plsc.cumsum(ok.astype(jnp.int32))
                        pos = oo + cs - 1
                        eid = ibase + go + lane
                        plsc.store_scatter(ebuf, [pos], eid, mask=ok)
                        plsc.store_scatter(abuf, [pos], local, mask=ok)
                        return oo + cs[15]

                    o = lax.cond(jnp.any(ok), compact, lambda oo: oo, o)
                return maybe_fire((o, fc))

            return lax.fori_loop(0, BIGC // 64, sub, state)

        pltpu.async_copy(idx_hbm.at[pl.ds(0, BIGC)], sv0, sc0)

        def big(ib, state):
            def proc(svb, scb, svo, sco, st):
                pltpu.make_async_copy(
                    idx_hbm.at[pl.ds(ib * BIGC, BIGC)], svb, scb).wait()

                @pl.when(ib + 1 < NBIG)
                def _():
                    pltpu.async_copy(
                        idx_hbm.at[pl.ds((ib + 1) * BIGC, BIGC)], svo, sco)

                return inner(svb, ib * BIGC, st)

            return lax.cond(
                ib % 2 == 0,
                lambda st: proc(sv0, sc0, sv1, sc1, st),
                lambda st: proc(sv1, sc1, sv0, sc0, st),
                state)

        off, fc = lax.fori_loop(0, NBIG, big, (0, 0))

        def drain0(_):
            pltpu.make_async_copy(vals_hbm.at[fid0], rows0, sg0).wait()
            accumulate(fadj0, rows0)
            return 0

        def drain1(_):
            pltpu.make_async_copy(vals_hbm.at[fid1], rows1, sg1).wait()
            accumulate(fadj1, rows1)
            return 0

        lax.cond(fc > 0,
                 lambda _: lax.cond((fc - 1) % 2 == 0, drain0, drain1, 0),
                 lambda _: 0, 0)

        @pl.when(fc >= 1)
        def _():
            pltpu.make_async_copy(fid0, le_hbm.at[wid, pl.ds(0, CF)], sl0).wait()
            pltpu.make_async_copy(fadj0, la_hbm.at[wid, pl.ds(0, CF)], sl0).wait()

        @pl.when(fc >= 2)
        def _():
            pltpu.make_async_copy(fid1, le_hbm.at[wid, pl.ds(0, CF)], sl1).wait()
            pltpu.make_async_copy(fadj1, la_hbm.at[wid, pl.ds(0, CF)], sl1).wait()

        for g in range(CF // 16):
            lm = (g * 16 + lane) < off
            fid0[pl.ds(g * 16, 16)] = jnp.where(lm, ebuf[pl.ds(g * 16, 16)], 0)
            fadj0[pl.ds(g * 16, 16)] = jnp.where(
                lm, abuf[pl.ds(g * 16, 16)], PAD_ROW)
        pltpu.sync_copy(fid0, le_hbm.at[wid, pl.ds(fc * CF, CF)])
        pltpu.sync_copy(fadj0, la_hbm.at[wid, pl.ds(fc * CF, CF)])
        cnt_v[pl.ds(0, 16)] = jnp.zeros((16,), jnp.int32) + (fc + 1)
        pltpu.sync_copy(cnt_v, cnt_hbm.at[wid])
        pltpu.async_copy(vals_hbm.at[fid0], rows0, sg0).wait()
        accumulate(fadj0, rows0)

        @pl.when(wid < NW - 1)
        def _():
            pltpu.sync_copy(acc_v.at[pl.ds(0, OWN)],
                            out_hbm.at[pl.ds(base_row, OWN)])

        @pl.when(wid == NW - 1)
        def _():
            pltpu.sync_copy(acc_v.at[pl.ds(0, OWN + 16)],
                            out_hbm.at[pl.ds(base_row, OWN + 16)])

    return k(vals, idx, zrows)


def _sc_scatter_from_lists(vals, le, la, cnts, zrows):
    mesh = plsc.VectorSubcoreMesh(core_axis_name="c", subcore_axis_name="s")

    @functools.partial(
        pl.kernel,
        out_type=jax.ShapeDtypeStruct((N, H), jnp.float32),
        mesh=mesh,
        compiler_params=pltpu.CompilerParams(needs_layout_passes=False),
        scratch_types=[
            pltpu.VMEM((CF,), jnp.int32),
            pltpu.VMEM((CF,), jnp.int32),
            pltpu.VMEM((CF,), jnp.int32),
            pltpu.VMEM((CF,), jnp.int32),
            pltpu.VMEM((CF, H), jnp.float32),
            pltpu.VMEM((CF, H), jnp.float32),
            pltpu.VMEM((ACC, H), jnp.float32),
            pltpu.VMEM((16,), jnp.int32),
            pltpu.SemaphoreType.DMA,
            pltpu.SemaphoreType.DMA,
            pltpu.SemaphoreType.DMA,
            pltpu.SemaphoreType.DMA,
        ],
    )
    def k(vals_hbm, le_hbm, la_hbm, cnt_hbm, z_hbm, out_hbm,
          fid0, fid1, fadj0, fadj1, rows0, rows1, acc_v, cnt_v,
          si0, si1, sg0, sg1):
        c = lax.axis_index("c")
        s = lax.axis_index("s")
        wid = s * NC + c
        base_row = wid * OWN

        for kz in range(8):
            pltpu.sync_copy(z_hbm, acc_v.at[pl.ds(kz * ZC, ZC)])
        pltpu.sync_copy(z_hbm.at[pl.ds(0, 16)], acc_v.at[pl.ds(320, 16)])

        pltpu.sync_copy(cnt_hbm.at[wid], cnt_v)
        nb = cnt_v[pl.ds(0, 16)][0]

        def accumulate(fadjb, rowsb):
            def grp(gg, carry):
                va = fadjb[pl.ds(gg * 16, 16)]
                for i in range(16):
                    r = gg * 16 + i
                    ld = va[i]
                    for j in range(H // 16):
                        plsc.addupdate(acc_v.at[ld, pl.ds(j * 16, 16)],
                                       rowsb[r, pl.ds(j * 16, 16)])
                return carry

            lax.fori_loop(0, CF // 16, grp, 0)

        pltpu.async_copy(le_hbm.at[wid, pl.ds(0, CF)], fid0, si0)
        pltpu.async_copy(la_hbm.at[wid, pl.ds(0, CF)], fadj0, si0)

        def run(j, fidb, fadjb, rowsb, sgb, sib, fido, fadjo, rowso, sgo, sio):
            pltpu.make_async_copy(
                le_hbm.at[wid, pl.ds(0, CF)], fidb, sib).wait()
            pltpu.make_async_copy(
                la_hbm.at[wid, pl.ds(0, CF)], fadjb, sib).wait()
            pltpu.async_copy(vals_hbm.at[fidb], rowsb, sgb)

            @pl.when(j >= 1)
            def _():
                pltpu.make_async_copy(vals_hbm.at[fido], rowso, sgo).wait()
                accumulate(fadjo, rowso)

            @pl.when(j + 1 < nb)
            def _():
                pltpu.async_copy(
                    le_hbm.at[wid, pl.ds((j + 1) * CF, CF)], fido, sio)
                pltpu.async_copy(
                    la_hbm.at[wid, pl.ds((j + 1) * CF, CF)], fadjo, sio)

        def body(j, carry):
            lax.cond(
                j % 2 == 0,
                lambda jj: run(jj, fid0, fadj0, rows0, sg0, si0,
                               fid1, fadj1, rows1, sg1, si1),
                lambda jj: run(jj, fid1, fadj1, rows1, sg1, si1,
                               fid0, fadj0, rows0, sg0, si0),
                j,
            )
            return carry

        lax.fori_loop(0, nb, body, 0)

        def drain0(_):
            pltpu.make_async_copy(vals_hbm.at[fid0], rows0, sg0).wait()
            accumulate(fadj0, rows0)
            return 0

        def drain1(_):
            pltpu.make_async_copy(vals_hbm.at[fid1], rows1, sg1).wait()
            accumulate(fadj1, rows1)
            return 0

        lax.cond((nb - 1) % 2 == 0, drain0, drain1, 0)

        @pl.when(wid < NW - 1)
        def _():
            pltpu.sync_copy(acc_v.at[pl.ds(0, OWN)],
                            out_hbm.at[pl.ds(base_row, OWN)])

        @pl.when(wid == NW - 1)
        def _():
            pltpu.sync_copy(acc_v.at[pl.ds(0, OWN + 16)],
                            out_hbm.at[pl.ds(base_row, OWN + 16)])

    return k(vals, le, la, cnts, zrows)


def kernel(x, edge_index, edge_attr, W_i, W_h, W_o, b_o, ln_g, ln_b):
    src = edge_index[0]
    dst = edge_index[1]
    wix = W_i[:, :ND].T
    wie = W_i[:, ND:].T
    wh = W_h.T
    wox = W_o[:, :ND].T
    wom = W_o[:, ND:].T
    g2d = ln_g.reshape(1, H)
    b2d = ln_b.reshape(1, H)
    bo2d = b_o.reshape(1, H)
    zrows = jnp.zeros((ZC, H), jnp.float32)

    xw = _tc_matmul(x, wix)
    g1 = _sc_gather(xw, src)
    msgs = _tc_msgs(g1, edge_attr, wie)
    agg, le, la, cnts = _sc_scatter_add(msgs, dst, zrows)
    aggw = _tc_matmul(agg, wh, relu=True)
    g2 = _sc_gather(aggw, src)
    new_msgs = _tc_ln(g2, msgs, g2d, b2d)
    nodemsg = _sc_scatter_from_lists(new_msgs, le, la, cnts, zrows)
    out = _tc_out(x, nodemsg, wox, wom, bo2d)
    return out, new_msgs


# speedup vs baseline: 1.0988x; 1.0988x over previous
import functools

import jax
import jax.numpy as jnp
from jax import lax
from jax.experimental import pallas as pl
from jax.experimental.pallas import tpu as pltpu
from jax.experimental.pallas import tpu_sc as plsc

N = 10000
E = 160000
ND = 256
ED = 16
H = 256

NC = 2
NS = 16
NW = NC * NS

C = 128
NCHUNK = E // C
HALF = N // NC
ZC = 40
DUM = 512


def _mm_kernel(a_ref, w_ref, o_ref):
    o_ref[...] = jnp.dot(a_ref[...], w_ref[...], preferred_element_type=jnp.float32)


def _mm_relu_kernel(a_ref, w_ref, o_ref):
    acc = jnp.dot(a_ref[...], w_ref[...], preferred_element_type=jnp.float32)
    o_ref[...] = jnp.maximum(acc, 0.0)


def _tc_matmul(a, w, relu=False, blk=2000, m=None):
    k = a.shape[1]
    m = a.shape[0] if m is None else m
    h = w.shape[1]
    return pl.pallas_call(
        _mm_relu_kernel if relu else _mm_kernel,
        grid=(m // blk,),
        in_specs=[pl.BlockSpec((blk, k), lambda i: (i, 0)),
                  pl.BlockSpec((k, h), lambda i: (0, 0))],
        out_specs=pl.BlockSpec((blk, h), lambda i: (i, 0)),
        out_shape=jax.ShapeDtypeStruct((m, h), jnp.float32),
    )(a, w)


def _msgs_kernel(g_ref, ea_ref, w_ref, o_ref):
    acc = jnp.dot(ea_ref[...], w_ref[...], preferred_element_type=jnp.float32)
    o_ref[...] = jnp.maximum(g_ref[...] + acc, 0.0)


def _tc_msgs(g, ea, w, blk=2000):
    return pl.pallas_call(
        _msgs_kernel,
        grid=(E // blk,),
        in_specs=[pl.BlockSpec((blk, H), lambda i: (i, 0)),
                  pl.BlockSpec((blk, ED), lambda i: (i, 0)),
                  pl.BlockSpec((ED, H), lambda i: (0, 0))],
        out_specs=pl.BlockSpec((blk, H), lambda i: (i, 0)),
        out_shape=jax.ShapeDtypeStruct((E, H), jnp.float32),
    )(g, ea, w)


def _ln_kernel(a_ref, b_ref, g_ref, bb_ref, o_ref):
    v = a_ref[...] + b_ref[...]
    mean = jnp.mean(v, axis=-1, keepdims=True)
    var = jnp.mean((v - mean) ** 2, axis=-1, keepdims=True)
    o_ref[...] = (v - mean) / jnp.sqrt(var + 1e-5) * g_ref[...] + bb_ref[...]


def _tc_ln(a, b, g, bb, blk=2000):
    return pl.pallas_call(
        _ln_kernel,
        grid=(E // blk,),
        in_specs=[pl.BlockSpec((blk, H), lambda i: (i, 0)),
                  pl.BlockSpec((blk, H), lambda i: (i, 0)),
                  pl.BlockSpec((1, H), lambda i: (0, 0)),
                  pl.BlockSpec((1, H), lambda i: (0, 0))],
        out_specs=pl.BlockSpec((blk, H), lambda i: (i, 0)),
        out_shape=jax.ShapeDtypeStruct((E, H), jnp.float32),
    )(a, b, g, bb)


def _out_kernel(x_ref, m_ref, wx_ref, wm_ref, b_ref, o_ref):
    acc = jnp.dot(x_ref[...], wx_ref[...], preferred_element_type=jnp.float32)
    acc = acc + jnp.dot(m_ref[...], wm_ref[...], preferred_element_type=jnp.float32)
    o_ref[...] = jnp.maximum(acc + b_ref[...], 0.0)


def _tc_out(x, m, wx, wm, b, blk=2000):
    return pl.pallas_call(
        _out_kernel,
        grid=(N // blk,),
        in_specs=[pl.BlockSpec((blk, ND), lambda i: (i, 0)),
                  pl.BlockSpec((blk, H), lambda i: (i, 0)),
                  pl.BlockSpec((ND, H), lambda i: (0, 0)),
                  pl.BlockSpec((H, H), lambda i: (0, 0)),
                  pl.BlockSpec((1, H), lambda i: (0, 0))],
        out_specs=pl.BlockSpec((blk, H), lambda i: (i, 0)),
        out_shape=jax.ShapeDtypeStruct((N, H), jnp.float32),
    )(x, m, wx, wm, b)


def _sc_gather(table, idx):
    mesh = plsc.VectorSubcoreMesh(core_axis_name="c", subcore_axis_name="s")
    base_cnt = NCHUNK // NW
    extra = NCHUNK - base_cnt * NW

    @functools.partial(
        pl.kernel,
        out_type=jax.ShapeDtypeStruct((E, H), jnp.float32),
        mesh=mesh,
        scratch_types=[
            pltpu.VMEM((C,), jnp.int32),
            pltpu.VMEM((C,), jnp.int32),
            pltpu.VMEM((C, H), jnp.float32),
            pltpu.VMEM((C, H), jnp.float32),
            pltpu.SemaphoreType.DMA,
            pltpu.SemaphoreType.DMA,
            pltpu.SemaphoreType.DMA,
            pltpu.SemaphoreType.DMA,
            pltpu.SemaphoreType.DMA,
            pltpu.SemaphoreType.DMA,
        ],
    )
    def k(table_hbm, idx_hbm, out_hbm, i0, i1, r0, r1,
          si0, si1, sg0, sg1, ss0, ss1):
        wid = lax.axis_index("s") * NC + lax.axis_index("c")
        cnt = base_cnt + (wid < extra).astype(jnp.int32)

        def cbase(j):
            return (wid + j * NW) * C

        pltpu.async_copy(idx_hbm.at[pl.ds(cbase(0), C)], i0, si0)

        def run_iter(j, ib, rb, sib, sgb, ssb, inxt, sinxt):
            pltpu.make_async_copy(idx_hbm.at[pl.ds(cbase(j), C)], ib, sib).wait()

            @pl.when(j >= 2)
            def _():
                pltpu.make_async_copy(
                    rb, out_hbm.at[pl.ds(cbase(j - 2), C)], ssb).wait()

            pltpu.async_copy(table_hbm.at[ib], rb, sgb)

            @pl.when(j + 1 < cnt)
            def _():
                pltpu.async_copy(
                    idx_hbm.at[pl.ds(cbase(j + 1), C)], inxt, sinxt)

            pltpu.make_async_copy(table_hbm.at[ib], rb, sgb).wait()
            pltpu.async_copy(rb, out_hbm.at[pl.ds(cbase(j), C)], ssb)

        def body(j, carry):
            lax.cond(
                j % 2 == 0,
                lambda jj: run_iter(jj, i0, r0, si0, sg0, ss0, i1, si1),
                lambda jj: run_iter(jj, i1, r1, si1, sg1, ss1, i0, si0),
                j,
            )
            return carry

        lax.fori_loop(0, cnt, body, 0)
        pltpu.make_async_copy(r0, out_hbm.at[pl.ds(0, C)], ss0).wait()
        pltpu.make_async_copy(r1, out_hbm.at[pl.ds(0, C)], ss1).wait()

    return k(table, idx)


OWN = 312
ACC = 336
PAD_ROW = ACC - 1
BIGC = 1600
NBIG = E // BIGC
CF = 64
PEND = 128


def _sc_scatter_add(vals, idx, zrows):
    mesh = plsc.VectorSubcoreMesh(core_axis_name="c", subcore_axis_name="s")

    @functools.partial(
        pl.kernel,
        out_type=(jax.ShapeDtypeStruct((N, H), jnp.float32),
                  jax.ShapeDtypeStruct((NW, E + CF), jnp.int32),
                  jax.ShapeDtypeStruct((NW, E + CF), jnp.int32),
                  jax.ShapeDtypeStruct((NW, 16), jnp.int32)),
        mesh=mesh,
        compiler_params=pltpu.CompilerParams(needs_layout_passes=False),
        scratch_types=[
            pltpu.VMEM((BIGC,), jnp.int32),
            pltpu.VMEM((BIGC,), jnp.int32),
            pltpu.VMEM((PEND,), jnp.int32),
            pltpu.VMEM((PEND,), jnp.int32),
            pltpu.VMEM((CF,), jnp.int32),
            pltpu.VMEM((CF,), jnp.int32),
            pltpu.VMEM((CF,), jnp.int32),
            pltpu.VMEM((CF,), jnp.int32),
            pltpu.VMEM((CF, H), jnp.float32),
            pltpu.VMEM((CF, H), jnp.float32),
            pltpu.VMEM((ACC, H), jnp.float32),
            pltpu.VMEM((16,), jnp.int32),
            pltpu.SemaphoreType.DMA,
            pltpu.SemaphoreType.DMA,
            pltpu.SemaphoreType.DMA,
            pltpu.SemaphoreType.DMA,
            pltpu.SemaphoreType.DMA,
            pltpu.SemaphoreType.DMA,
        ],
    )
    def k(vals_hbm, idx_hbm, z_hbm, out_hbm, le_hbm, la_hbm, cnt_hbm,
          sv0, sv1, ebuf, abuf, fid0, fid1, fadj0, fadj1, rows0, rows1,
          acc_v, cnt_v, sc0, sc1, sg0, sg1, sl0, sl1):
        c = lax.axis_index("c")
        s = lax.axis_index("s")
        wid = s * NC + c
        lane = lax.iota(jnp.int32, 16)
        base_row = wid * OWN
        ub = OWN + jnp.where(wid == NW - 1, ACC - OWN - 8, 0)

        for kz in range(8):
            pltpu.sync_copy(z_hbm, acc_v.at[pl.ds(kz * ZC, ZC)])
        pltpu.sync_copy(z_hbm.at[pl.ds(0, 16)], acc_v.at[pl.ds(320, 16)])

        def accumulate(fadjb, rowsb):
            def grp(gg, carry):
                va = fadjb[pl.ds(gg * 16, 16)]
                for i in range(16):
                    r = gg * 16 + i
                    ld = va[i]
                    for j in range(H // 16):
                        plsc.addupdate(acc_v.at[ld, pl.ds(j * 16, 16)],
                                       rowsb[r, pl.ds(j * 16, 16)])
                return carry

            lax.fori_loop(0, CF // 16, grp, 0)

        def fire_buf(o, fc, fidb, fadjb, rowsb, sgb, slb,
                     fido, fadjo, rowso, sgo):
            @pl.when(fc >= 2)
            def _():
                pltpu.make_async_copy(
                    fidb, le_hbm.at[wid, pl.ds(0, CF)], slb).wait()
                pltpu.make_async_copy(
                    fadjb, la_hbm.at[wid, pl.ds(0, CF)], slb).wait()

            for g in range(CF // 16):
                fidb[pl.ds(g * 16, 16)] = ebuf[pl.ds(g * 16, 16)]
                fadjb[pl.ds(g * 16, 16)] = abuf[pl.ds(g * 16, 16)]
            for g in range(CF // 16):
                ebuf[pl.ds(g * 16, 16)] = ebuf[pl.ds(CF + g * 16, 16)]
                abuf[pl.ds(g * 16, 16)] = abuf[pl.ds(CF + g * 16, 16)]

            pltpu.async_copy(fidb, le_hbm.at[wid, pl.ds(fc * CF, CF)], slb)
            pltpu.async_copy(fadjb, la_hbm.at[wid, pl.ds(fc * CF, CF)], slb)

            @pl.when(fc > 0)
            def _():
                pltpu.make_async_copy(vals_hbm.at[fido], rowso, sgo).wait()
                accumulate(fadjo, rowso)

            pltpu.async_copy(vals_hbm.at[fidb], rowsb, sgb)
            return (o - CF, fc + 1)

        def fire(state):
            return lax.cond(
                state[1] % 2 == 0,
                lambda st: fire_buf(st[0], st[1], fid0, fadj0, rows0, sg0, sl0,
                                    fid1, fadj1, rows1, sg1),
                lambda st: fire_buf(st[0], st[1], fid1, fadj1, rows1, sg1, sl1,
                                    fid0, fadj0, rows0, sg0),
                state)

        def maybe_fire(state):
            return lax.cond(state[0] >= CF, fire, lambda st: st, state)

        def inner(svb, ibase, state):
            def sub(gi, st):
                o, fc = st
                for g in range(4):
                    go = gi * 64 + g * 16
                    v = svb[pl.ds(go, 16)]
                    local = v - base_row
                    ok = (local >= 0) & (local < ub)
                    cs = plsc.cumsum(ok.astype(jnp.int32))
                    pos = o + cs - 1
                    eid = ibase + go + lane
                    plsc.store_scatter(ebuf, [pos], eid, mask=ok)
                    plsc.store_scatter(abuf, [pos], local, mask=ok)
                    o = o + cs[15]
                return maybe_fire((o, fc))

            return lax.fori_loop(0, BIGC // 64, sub, state)

        pltpu.async_copy(idx_hbm.at[pl.ds(0, BIGC)], sv0, sc0)

        def big(ib, state):
            def proc(svb, scb, svo, sco, st):
                pltpu.make_async_copy(
                    idx_hbm.at[pl.ds(ib * BIGC, BIGC)], svb, scb).wait()

                @pl.when(ib + 1 < NBIG)
                def _():
                    pltpu.async_copy(
                        idx_hbm.at[pl.ds((ib + 1) * BIGC, BIGC)], svo, sco)

                return inner(svb, ib * BIGC, st)

            return lax.cond(
                ib % 2 == 0,
                lambda st: proc(sv0, sc0, sv1, sc1, st),
                lambda st: proc(sv1, sc1, sv0, sc0, st),
                state)

        off, fc = lax.fori_loop(0, NBIG, big, (0, 0))

        def drain0(_):
            pltpu.make_async_copy(vals_hbm.at[fid0], rows0, sg0).wait()
            accumulate(fadj0, rows0)
            return 0

        def drain1(_):
            pltpu.make_async_copy(vals_hbm.at[fid1], rows1, sg1).wait()
            accumulate(fadj1, rows1)
            return 0

        lax.cond(fc > 0,
                 lambda _: lax.cond((fc - 1) % 2 == 0, drain0, drain1, 0),
                 lambda _: 0, 0)

        @pl.when(fc >= 1)
        def _():
            pltpu.make_async_copy(fid0, le_hbm.at[wid, pl.ds(0, CF)], sl0).wait()
            pltpu.make_async_copy(fadj0, la_hbm.at[wid, pl.ds(0, CF)], sl0).wait()

        @pl.when(fc >= 2)
        def _():
            pltpu.make_async_copy(fid1, le_hbm.at[wid, pl.ds(0, CF)], sl1).wait()
            pltpu.make_async_copy(fadj1, la_hbm.at[wid, pl.ds(0, CF)], sl1).wait()

        for g in range(CF // 16):
            lm = (g * 16 + lane) < off
            fid0[pl.ds(g * 16, 16)] = jnp.where(lm, ebuf[pl.ds(g * 16, 16)], 0)
            fadj0[pl.ds(g * 16, 16)] = jnp.where(
                lm, abuf[pl.ds(g * 16, 16)], PAD_ROW)
        pltpu.sync_copy(fid0, le_hbm.at[wid, pl.ds(fc * CF, CF)])
        pltpu.sync_copy(fadj0, la_hbm.at[wid, pl.ds(fc * CF, CF)])
        cnt_v[pl.ds(0, 16)] = jnp.zeros((16,), jnp.int32) + (fc + 1)
        pltpu.sync_copy(cnt_v, cnt_hbm.at[wid])
        pltpu.async_copy(vals_hbm.at[fid0], rows0, sg0).wait()
        accumulate(fadj0, rows0)

        @pl.when(wid < NW - 1)
        def _():
            pltpu.sync_copy(acc_v.at[pl.ds(0, OWN)],
                            out_hbm.at[pl.ds(base_row, OWN)])

        @pl.when(wid == NW - 1)
        def _():
            pltpu.sync_copy(acc_v.at[pl.ds(0, OWN + 16)],
                            out_hbm.at[pl.ds(base_row, OWN + 16)])

    return k(vals, idx, zrows)


def _sc_scatter_from_lists(vals, le, la, cnts, zrows):
    mesh = plsc.VectorSubcoreMesh(core_axis_name="c", subcore_axis_name="s")

    @functools.partial(
        pl.kernel,
        out_type=jax.ShapeDtypeStruct((N, H), jnp.float32),
        mesh=mesh,
        compiler_params=pltpu.CompilerParams(needs_layout_passes=False),
        scratch_types=[
            pltpu.VMEM((CF,), jnp.int32),
            pltpu.VMEM((CF,), jnp.int32),
            pltpu.VMEM((CF,), jnp.int32),
            pltpu.VMEM((CF,), jnp.int32),
            pltpu.VMEM((CF, H), jnp.float32),
            pltpu.VMEM((CF, H), jnp.float32),
            pltpu.VMEM((ACC, H), jnp.float32),
            pltpu.VMEM((16,), jnp.int32),
            pltpu.SemaphoreType.DMA,
            pltpu.SemaphoreType.DMA,
            pltpu.SemaphoreType.DMA,
            pltpu.SemaphoreType.DMA,
        ],
    )
    def k(vals_hbm, le_hbm, la_hbm, cnt_hbm, z_hbm, out_hbm,
          fid0, fid1, fadj0, fadj1, rows0, rows1, acc_v, cnt_v,
          si0, si1, sg0, sg1):
        c = lax.axis_index("c")
        s = lax.axis_index("s")
        wid = s * NC + c
        base_row = wid * OWN

        for kz in range(8):
            pltpu.sync_copy(z_hbm, acc_v.at[pl.ds(kz * ZC, ZC)])
        pltpu.sync_copy(z_hbm.at[pl.ds(0, 16)], acc_v.at[pl.ds(320, 16)])

        pltpu.sync_copy(cnt_hbm.at[wid], cnt_v)
        nb = cnt_v[pl.ds(0, 16)][0]

        def accumulate(fadjb, rowsb):
            def grp(gg, carry):
                va = fadjb[pl.ds(gg * 16, 16)]
                for i in range(16):
                    r = gg * 16 + i
                    ld = va[i]
                    for j in range(H // 16):
                        plsc.addupdate(acc_v.at[ld, pl.ds(j * 16, 16)],
                                       rowsb[r, pl.ds(j * 16, 16)])
                return carry

            lax.fori_loop(0, CF // 16, grp, 0)

        pltpu.async_copy(le_hbm.at[wid, pl.ds(0, CF)], fid0, si0)
        pltpu.async_copy(la_hbm.at[wid, pl.ds(0, CF)], fadj0, si0)

        def run(j, fidb, fadjb, rowsb, sgb, sib, fido, fadjo, rowso, sgo, sio):
            pltpu.make_async_copy(
                le_hbm.at[wid, pl.ds(0, CF)], fidb, sib).wait()
            pltpu.make_async_copy(
                la_hbm.at[wid, pl.ds(0, CF)], fadjb, sib).wait()
            pltpu.async_copy(vals_hbm.at[fidb], rowsb, sgb)

            @pl.when(j >= 1)
            def _():
                pltpu.make_async_copy(vals_hbm.at[fido], rowso, sgo).wait()
                accumulate(fadjo, rowso)

            @pl.when(j + 1 < nb)
            def _():
                pltpu.async_copy(
                    le_hbm.at[wid, pl.ds((j + 1) * CF, CF)], fido, sio)
                pltpu.async_copy(
                    la_hbm.at[wid, pl.ds((j + 1) * CF, CF)], fadjo, sio)

        def body(j, carry):
            lax.cond(
                j % 2 == 0,
                lambda jj: run(jj, fid0, fadj0, rows0, sg0, si0,
                               fid1, fadj1, rows1, sg1, si1),
                lambda jj: run(jj, fid1, fadj1, rows1, sg1, si1,
                               fid0, fadj0, rows0, sg0, si0),
                j,
            )
            return carry

        lax.fori_loop(0, nb, body, 0)

        def drain0(_):
            pltpu.make_async_copy(vals_hbm.at[fid0], rows0, sg0).wait()
            accumulate(fadj0, rows0)
            return 0

        def drain1(_):
            pltpu.make_async_copy(vals_hbm.at[fid1], rows1, sg1).wait()
            accumulate(fadj1, rows1)
            return 0

        lax.cond((nb - 1) % 2 == 0, drain0, drain1, 0)

        @pl.when(wid < NW - 1)
        def _():
            pltpu.sync_copy(acc_v.at[pl.ds(0, OWN)],
                            out_hbm.at[pl.ds(base_row, OWN)])

        @pl.when(wid == NW - 1)
        def _():
            pltpu.sync_copy(acc_v.at[pl.ds(0, OWN + 16)],
                            out_hbm.at[pl.ds(base_row, OWN + 16)])

    return k(vals, le, la, cnts, zrows)


def kernel(x, edge_index, edge_attr, W_i, W_h, W_o, b_o, ln_g, ln_b):
    src = edge_index[0]
    dst = edge_index[1]
    wix = W_i[:, :ND].T
    wie = W_i[:, ND:].T
    wh = W_h.T
    wox = W_o[:, :ND].T
    wom = W_o[:, ND:].T
    g2d = ln_g.reshape(1, H)
    b2d = ln_b.reshape(1, H)
    bo2d = b_o.reshape(1, H)
    zrows = jnp.zeros((ZC, H), jnp.float32)

    xw = _tc_matmul(x, wix)
    g1 = _sc_gather(xw, src)
    msgs = _tc_msgs(g1, edge_attr, wie)
    agg, le, la, cnts = _sc_scatter_add(msgs, dst, zrows)
    aggw = _tc_matmul(agg, wh, relu=True)
    g2 = _sc_gather(aggw, src)
    new_msgs = _tc_ln(g2, msgs, g2d, b2d)
    nodemsg = _sc_scatter_from_lists(new_msgs, le, la, cnts, zrows)
    out = _tc_out(x, nodemsg, wox, wom, bo2d)
    return out, new_msgs
